# initial kernel scaffold (unmeasured)
import jax
import jax.numpy as jnp
from jax import lax
from jax.experimental import pallas as pl
from jax.experimental.pallas import tpu as pltpu

N_DEV = 8
NPASS = 2


def kernel(x, w_mat):
    M, Kp = x.shape
    _, N = w_mat.shape
    CH = M // N_DEV
    NH = N // NPASS

    def body(x_ref, w_ref, out_ref, b_ref, c_ref, amax_rbuf, amax_sbuf,
             send_sems, recv_sems, a_send_sems, a_recv_sems, dma_sems,
             credit):
        d = lax.axis_index("i")
        right = (d + 1) % N_DEV
        left = (d + N_DEV - 1) % N_DEV

        barrier = pltpu.get_barrier_semaphore()
        for nbr in (left, right):
            pl.semaphore_signal(barrier, 1, device_id=(nbr,))
        pl.semaphore_wait(barrier, 2)

        amax_rbuf[...] = jnp.zeros((N_DEV, 128), jnp.float32)

        def gemm_chunk(cidx, p):
            return jnp.dot(
                x_ref[pl.ds(cidx * CH, CH), :],
                w_ref[:, p * NH:(p + 1) * NH],
                preferred_element_type=jnp.float32,
            )

        def credit_left():
            pl.semaphore_signal(credit, 1, device_id=(left,))

        def ring_hop(buf, s_slot, r_slot, first):
            if not first:
                pl.semaphore_wait(credit, 1)
            rdma = pltpu.make_async_remote_copy(
                src_ref=buf.at[s_slot],
                dst_ref=buf.at[r_slot],
                send_sem=send_sems.at[s_slot],
                recv_sem=recv_sems.at[r_slot],
                device_id=(right,),
            )
            rdma.start()
            rdma.wait()

        def stage_out(slot, rows_at, col_pass):
            cp = pltpu.make_async_copy(
                b_ref.at[slot],
                out_ref.at[pl.ds(rows_at * CH, CH),
                           pl.ds(col_pass * NH, NH)],
                dma_sems.at[slot],
            )
            cp.start()
            cp.wait()

        amax_local = jnp.float32(0.0)
        k = 0
        b_ref[0] = gemm_chunk((d + N_DEV - 1) % N_DEV, 0)
        for p in range(NPASS):
            for h in range(N_DEV - 1):
                s_slot, r_slot = k % 2, (k + 1) % 2
                ring_hop(b_ref, s_slot, r_slot, first=(k == 0))
                if h < N_DEV - 2:
                    c_recv = (d + 2 * N_DEV - h - 2) % N_DEV
                    b_ref[r_slot] = b_ref[r_slot] + gemm_chunk(c_recv, p)
                else:
                    fin = jnp.maximum(b_ref[r_slot] + gemm_chunk(d, p), 0.0)
                    b_ref[r_slot] = fin
                    amax_local = jnp.maximum(amax_local, jnp.max(fin))
                    stage_out(r_slot, d, p)
                    if p < NPASS - 1:
                        b_ref[r_slot] = gemm_chunk((d + N_DEV - 1) % N_DEV,
                                                   p + 1)
                credit_left()
                k += 1

        amax_sbuf[...] = jnp.full((1, 128), amax_local, jnp.float32)
        sends = []
        for off in range(1, N_DEV):
            tgt = (d + off) % N_DEV
            rd = pltpu.make_async_remote_copy(
                src_ref=amax_sbuf,
                dst_ref=amax_rbuf.at[pl.ds(d, 1)],
                send_sem=a_send_sems.at[off],
                recv_sem=a_recv_sems.at[d],
                device_id=(tgt,),
            )
            rd.start()
            sends.append(rd)
        for off in range(1, N_DEV):
            src = (d + off) % N_DEV
            rr = pltpu.make_async_remote_copy(
                src_ref=amax_sbuf,
                dst_ref=amax_rbuf.at[pl.ds(src, 1)],
                send_sem=a_send_sems.at[0],
                recv_sem=a_recv_sems.at[src],
                device_id=(left,),
            )
            rr.wait_recv()
        for rd in sends:
            rd.wait_send()
        amax_g = jnp.maximum(jnp.max(amax_rbuf[...]), amax_local)
        inv = 127.0 / amax_g
        scale = amax_g / 127.0

        for p in range(NPASS):
            cp = pltpu.make_async_copy(
                out_ref.at[pl.ds(d * CH, CH), pl.ds(p * NH, NH)],
                b_ref.at[p % 2],
                dma_sems.at[p % 2],
            )
            cp.start()
            cp.wait()
            q = jnp.clip(jnp.round(b_ref[p % 2] * inv), -127.0, 127.0)
            c_ref[0, :, p * NH:(p + 1) * NH] = q.astype(jnp.int8)
        for p in range(NPASS):
            b_ref[p % 2] = (
                c_ref[0, :, p * NH:(p + 1) * NH].astype(jnp.float32) * scale
            )
            stage_out(p % 2, d, p)

        for a in range(N_DEV - 1):
            s_slot, r_slot = a % 2, (a + 1) % 2
            pl.semaphore_wait(credit, 1)
            rdma = pltpu.make_async_remote_copy(
                src_ref=c_ref.at[s_slot],
                dst_ref=c_ref.at[r_slot],
                send_sem=send_sems.at[s_slot],
                recv_sem=recv_sems.at[r_slot],
                device_id=(right,),
            )
            rdma.start()
            rdma.wait()
            origin = (d + 2 * N_DEV - a - 1) % N_DEV
            for p in range(NPASS):
                b_ref[p % 2] = (
                    c_ref[r_slot, :, p * NH:(p + 1) * NH].astype(jnp.float32)
                    * scale
                )
                stage_out(p % 2, origin, p)
            credit_left()
        pl.semaphore_wait(credit, 1)

    return pl.pallas_call(
        body,
        out_shape=jax.ShapeDtypeStruct((M, N), jnp.float32),
        in_specs=[
            pl.BlockSpec(memory_space=pltpu.VMEM),
            pl.BlockSpec(memory_space=pltpu.VMEM),
        ],
        out_specs=pl.BlockSpec(memory_space=pltpu.ARBITRARY),
        scratch_shapes=[
            pltpu.VMEM((2, M // N_DEV, N // NPASS), jnp.float32),
            pltpu.VMEM((2, M // N_DEV, N), jnp.int8),
            pltpu.VMEM((N_DEV, 128), jnp.float32),
            pltpu.VMEM((1, 128), jnp.float32),
            pltpu.SemaphoreType.DMA((2,)),
            pltpu.SemaphoreType.DMA((2,)),
            pltpu.SemaphoreType.DMA((N_DEV,)),
            pltpu.SemaphoreType.DMA((N_DEV,)),
            pltpu.SemaphoreType.DMA((2,)),
            pltpu.SemaphoreType.REGULAR,
        ],
        compiler_params=pltpu.CompilerParams(
            collective_id=0,
            vmem_limit_bytes=64 * 1024 * 1024,
        ),
    )(x, w_mat)


# baseline (device time: 1866135 ns/iter reference)
import jax
import jax.numpy as jnp
from jax import lax
from jax.experimental import pallas as pl
from jax.experimental.pallas import tpu as pltpu

N_DEV = 8
NPASS = 2


def kernel(x, w_mat):
    M, Kp = x.shape
    _, N = w_mat.shape
    CH = M // N_DEV
    NH = N // NPASS

    def body(x_ref, w_ref, out_ref, b_ref, c_ref, amax_rbuf, amax_sbuf,
             send_sems, recv_sems, a_send_sems, a_recv_sems, dma_sems,
             credit):
        d = lax.axis_index("i")
        right = (d + 1) % N_DEV
        left = (d + N_DEV - 1) % N_DEV

        barrier = pltpu.get_barrier_semaphore()
        for nbr in (left, right):
            pl.semaphore_signal(barrier, 1, device_id=(nbr,))
        pl.semaphore_wait(barrier, 2)

        amax_rbuf[...] = jnp.zeros((N_DEV, 128), jnp.float32)

        def gemm_chunk(cidx, p):
            return jnp.dot(
                x_ref[pl.ds(cidx * CH, CH), :],
                w_ref[:, p * NH:(p + 1) * NH],
                preferred_element_type=jnp.float32,
            )

        def credit_left():
            pl.semaphore_signal(credit, 1, device_id=(left,))

        def ring_hop(buf, s_slot, r_slot, first):
            if not first:
                pl.semaphore_wait(credit, 1)
            rdma = pltpu.make_async_remote_copy(
                src_ref=buf.at[s_slot],
                dst_ref=buf.at[r_slot],
                send_sem=send_sems.at[s_slot],
                recv_sem=recv_sems.at[r_slot],
                device_id=(right,),
            )
            rdma.start()
            rdma.wait()

        def stage_out(slot, rows_at, col_pass):
            cp = pltpu.make_async_copy(
                b_ref.at[slot],
                out_ref.at[pl.ds(rows_at * CH, CH),
                           pl.ds(col_pass * NH, NH)],
                dma_sems.at[slot],
            )
            cp.start()
            cp.wait()

        amax_local = jnp.float32(0.0)
        k = 0
        b_ref[0] = gemm_chunk((d + N_DEV - 1) % N_DEV, 0)
        for p in range(NPASS):
            for h in range(N_DEV - 1):
                s_slot, r_slot = k % 2, (k + 1) % 2
                ring_hop(b_ref, s_slot, r_slot, first=(k == 0))
                if h < N_DEV - 2:
                    c_recv = (d + 2 * N_DEV - h - 2) % N_DEV
                    b_ref[r_slot] = b_ref[r_slot] + gemm_chunk(c_recv, p)
                else:
                    fin = jnp.maximum(b_ref[r_slot] + gemm_chunk(d, p), 0.0)
                    b_ref[r_slot] = fin
                    amax_local = jnp.maximum(amax_local, jnp.max(fin))
                    stage_out(r_slot, d, p)
                    if p < NPASS - 1:
                        b_ref[r_slot] = gemm_chunk((d + N_DEV - 1) % N_DEV,
                                                   p + 1)
                credit_left()
                k += 1

        amax_sbuf[...] = jnp.full((1, 128), amax_local, jnp.float32)
        sends = []
        for off in range(1, N_DEV):
            tgt = (d + off) % N_DEV
            rd = pltpu.make_async_remote_copy(
                src_ref=amax_sbuf,
                dst_ref=amax_rbuf.at[pl.ds(d, 1)],
                send_sem=a_send_sems.at[off],
                recv_sem=a_recv_sems.at[d],
                device_id=(tgt,),
            )
            rd.start()
            sends.append(rd)
        for off in range(1, N_DEV):
            src = (d + off) % N_DEV
            rr = pltpu.make_async_remote_copy(
                src_ref=amax_sbuf,
                dst_ref=amax_rbuf.at[pl.ds(src, 1)],
                send_sem=a_send_sems.at[0],
                recv_sem=a_recv_sems.at[src],
                device_id=(left,),
            )
            rr.wait_recv()
        for rd in sends:
            rd.wait_send()
        amax_g = jnp.maximum(jnp.max(amax_rbuf[...]), amax_local)
        inv = 127.0 / amax_g
        scale = amax_g / 127.0

        for p in range(NPASS):
            cp = pltpu.make_async_copy(
                out_ref.at[pl.ds(d * CH, CH), pl.ds(p * NH, NH)],
                b_ref.at[p % 2],
                dma_sems.at[p % 2],
            )
            cp.start()
            cp.wait()
            q = jnp.clip(jnp.round(b_ref[p % 2] * inv), -127.0, 127.0)
            c_ref[0, :, p * NH:(p + 1) * NH] = q.astype(jnp.int8)
        for p in range(NPASS):
            b_ref[p % 2] = (
                c_ref[0, :, p * NH:(p + 1) * NH].astype(jnp.float32) * scale
            )
            stage_out(p % 2, d, p)

        for a in range(N_DEV - 1):
            s_slot, r_slot = a % 2, (a + 1) % 2
            pl.semaphore_wait(credit, 1)
            rdma = pltpu.make_async_remote_copy(
                src_ref=c_ref.at[s_slot],
                dst_ref=c_ref.at[r_slot],
                send_sem=send_sems.at[s_slot],
                recv_sem=recv_sems.at[r_slot],
                device_id=(right,),
            )
            rdma.start()
            rdma.wait()
            origin = (d + 2 * N_DEV - a - 1) % N_DEV
            for p in range(NPASS):
                b_ref[p % 2] = (
                    c_ref[r_slot, :, p * NH:(p + 1) * NH].astype(jnp.float32)
                    * scale
                )
                stage_out(p % 2, origin, p)
            credit_left()
        pl.semaphore_wait(credit, 1)

    return pl.pallas_call(
        body,
        out_shape=jax.ShapeDtypeStruct((M, N), jnp.float32),
        in_specs=[
            pl.BlockSpec(memory_space=pltpu.VMEM),
            pl.BlockSpec(memory_space=pltpu.VMEM),
        ],
        out_specs=pl.BlockSpec(memory_space=pl.ANY),
        scratch_shapes=[
            pltpu.VMEM((2, M // N_DEV, N // NPASS), jnp.float32),
            pltpu.VMEM((2, M // N_DEV, N), jnp.int8),
            pltpu.VMEM((N_DEV, 128), jnp.float32),
            pltpu.VMEM((1, 128), jnp.float32),
            pltpu.SemaphoreType.DMA((2,)),
            pltpu.SemaphoreType.DMA((2,)),
            pltpu.SemaphoreType.DMA((N_DEV,)),
            pltpu.SemaphoreType.DMA((N_DEV,)),
            pltpu.SemaphoreType.DMA((2,)),
            pltpu.SemaphoreType.REGULAR,
        ],
        compiler_params=pltpu.CompilerParams(
            collective_id=0,
            vmem_limit_bytes=64 * 1024 * 1024,
        ),
    )(x, w_mat)


# device time: 1072419 ns/iter; 1.7401x vs baseline; 1.7401x over previous
import jax
import jax.numpy as jnp
from jax import lax
from jax.experimental import pallas as pl
from jax.experimental.pallas import tpu as pltpu

N_DEV = 8
NPASS = 2


def kernel(x, w_mat):
    M, Kp = x.shape
    _, N = w_mat.shape
    CH = M // N_DEV
    NH = N // NPASS
    HW = NH // 2
    QW = N // 4
    AGW = N // 2

    def body(x_ref, w_ref, out_ref,
             bcw, bccw, ccw_i8, cccw_i8, amax_rbuf, amax_sbuf,
             snd_cw, rcv_cw, snd_ccw, rcv_ccw,
             a_snd, a_rcv, dma_sems, cred_cw, cred_ccw):
        d = lax.axis_index("i")
        right = (d + 1) % N_DEV
        left = (d + N_DEV - 1) % N_DEV

        barrier = pltpu.get_barrier_semaphore()
        for nbr in (left, right):
            pl.semaphore_signal(barrier, 1, device_id=(nbr,))
        pl.semaphore_wait(barrier, 2)

        amax_rbuf[...] = jnp.zeros((N_DEV, 128), jnp.float32)

        def gemm_cols(cidx, c0, width):
            return jnp.dot(
                x_ref[pl.ds(cidx * CH, CH), :],
                w_ref[:, c0:c0 + width],
                preferred_element_type=jnp.float32,
            )

        def hop_pair(buf_cw, buf_ccw, s_slot, r_slot, first):
            if not first:
                pl.semaphore_wait(cred_cw, 1)
                pl.semaphore_wait(cred_ccw, 1)
            r1 = pltpu.make_async_remote_copy(
                src_ref=buf_cw.at[s_slot], dst_ref=buf_cw.at[r_slot],
                send_sem=snd_cw.at[s_slot], recv_sem=rcv_cw.at[r_slot],
                device_id=(right,),
            )
            r2 = pltpu.make_async_remote_copy(
                src_ref=buf_ccw.at[s_slot], dst_ref=buf_ccw.at[r_slot],
                send_sem=snd_ccw.at[s_slot], recv_sem=rcv_ccw.at[r_slot],
                device_id=(left,),
            )
            r1.start()
            r2.start()
            r1.wait()
            r2.wait()

        def credits():
            pl.semaphore_signal(cred_cw, 1, device_id=(left,))
            pl.semaphore_signal(cred_ccw, 1, device_id=(right,))

        def out_dma(buf, slot, rows_at, c0, width, sem_i):
            return pltpu.make_async_copy(
                buf.at[slot],
                out_ref.at[pl.ds(rows_at * CH, CH), pl.ds(c0, width)],
                dma_sems.at[sem_i],
            )

        amax_local = jnp.float32(0.0)
        k = 0
        bcw[0] = gemm_cols((d + N_DEV - 1) % N_DEV, 0, HW)
        bccw[0] = gemm_cols((d + 1) % N_DEV, HW, HW)
        for p in range(NPASS):
            cw0 = p * NH
            ccw0 = p * NH + HW
            for h in range(N_DEV - 1):
                s_slot, r_slot = k % 2, (k + 1) % 2
                hop_pair(bcw, bccw, s_slot, r_slot, first=(k == 0))
                if h < N_DEV - 2:
                    c_cw = (d + 2 * N_DEV - h - 2) % N_DEV
                    c_ccw = (d + h + 2) % N_DEV
                    bcw[r_slot] = bcw[r_slot] + gemm_cols(c_cw, cw0, HW)
                    bccw[r_slot] = bccw[r_slot] + gemm_cols(c_ccw, ccw0, HW)
                else:
                    fcw = jnp.maximum(bcw[r_slot] + gemm_cols(d, cw0, HW),
                                      0.0)
                    fccw = jnp.maximum(bccw[r_slot] + gemm_cols(d, ccw0, HW),
                                       0.0)
                    bcw[r_slot] = fcw
                    bccw[r_slot] = fccw
                    amax_local = jnp.maximum(
                        amax_local,
                        jnp.maximum(jnp.max(fcw), jnp.max(fccw)))
                    cp1 = out_dma(bcw, r_slot, d, cw0, HW, 0)
                    cp2 = out_dma(bccw, r_slot, d, ccw0, HW, 1)
                    cp1.start()
                    cp2.start()
                    cp1.wait()
                    cp2.wait()
                    if p < NPASS - 1:
                        bcw[r_slot] = gemm_cols((d + N_DEV - 1) % N_DEV,
                                                (p + 1) * NH, HW)
                        bccw[r_slot] = gemm_cols((d + 1) % N_DEV,
                                                 (p + 1) * NH + HW, HW)
                credits()
                k += 1

        amax_sbuf[...] = jnp.full((1, 128), amax_local, jnp.float32)
        sends = []
        for off in range(1, N_DEV):
            tgt = (d + off) % N_DEV
            rd = pltpu.make_async_remote_copy(
                src_ref=amax_sbuf,
                dst_ref=amax_rbuf.at[pl.ds(d, 1)],
                send_sem=a_snd.at[off],
                recv_sem=a_rcv.at[d],
                device_id=(tgt,),
            )
            rd.start()
            sends.append(rd)
        for off in range(1, N_DEV):
            src = (d + off) % N_DEV
            rr = pltpu.make_async_remote_copy(
                src_ref=amax_sbuf,
                dst_ref=amax_rbuf.at[pl.ds(src, 1)],
                send_sem=a_snd.at[0],
                recv_sem=a_rcv.at[src],
                device_id=(left,),
            )
            rr.wait_recv()
        for rd in sends:
            rd.wait_send()
        amax_g = jnp.maximum(jnp.max(amax_rbuf[...]), amax_local)
        inv = 127.0 / amax_g
        scale = amax_g / 127.0

        quarters = ((bcw, 0), (bcw, 1), (bccw, 0), (bccw, 1))
        for qi, (buf, slot) in enumerate(quarters):
            cp = pltpu.make_async_copy(
                out_ref.at[pl.ds(d * CH, CH), pl.ds(qi * QW, QW)],
                buf.at[slot],
                dma_sems.at[qi],
            )
            cp.start()
            cp.wait()
            q = jnp.clip(jnp.round(buf[slot] * inv), -127.0, 127.0)
            cref, off = (ccw_i8, qi) if qi < 2 else (cccw_i8, qi - 2)
            cref[0, :, off * QW:(off + 1) * QW] = q.astype(jnp.int8)
        dmas = []
        for qi, (buf, slot) in enumerate(quarters):
            cref, off = (ccw_i8, qi) if qi < 2 else (cccw_i8, qi - 2)
            buf[slot] = (
                cref[0, :, off * QW:(off + 1) * QW].astype(jnp.float32)
                * scale
            )
            cp = out_dma(buf, slot, d, qi * QW, QW, qi)
            cp.start()
            dmas.append(cp)
        for cp in dmas:
            cp.wait()

        for a in range(N_DEV - 1):
            s_slot, r_slot = a % 2, (a + 1) % 2
            hop_pair(ccw_i8, cccw_i8, s_slot, r_slot, first=False)
            o_cw = (d + 2 * N_DEV - a - 1) % N_DEV
            o_ccw = (d + a + 1) % N_DEV
            dmas = []
            for qi, (buf, slot) in enumerate(quarters):
                cref, off = (ccw_i8, qi) if qi < 2 else (cccw_i8, qi - 2)
                origin = o_cw if qi < 2 else o_ccw
                buf[slot] = (
                    cref[r_slot, :, off * QW:(off + 1) * QW]
                    .astype(jnp.float32) * scale
                )
                cp = out_dma(buf, slot, origin, qi * QW, QW, qi)
                cp.start()
                dmas.append(cp)
            for cp in dmas:
                cp.wait()
            credits()
        pl.semaphore_wait(cred_cw, 1)
        pl.semaphore_wait(cred_ccw, 1)

    return pl.pallas_call(
        body,
        out_shape=jax.ShapeDtypeStruct((M, N), jnp.float32),
        in_specs=[
            pl.BlockSpec(memory_space=pltpu.VMEM),
            pl.BlockSpec(memory_space=pltpu.VMEM),
        ],
        out_specs=pl.BlockSpec(memory_space=pl.ANY),
        scratch_shapes=[
            pltpu.VMEM((2, M // N_DEV, NH // 2), jnp.float32),
            pltpu.VMEM((2, M // N_DEV, NH // 2), jnp.float32),
            pltpu.VMEM((2, M // N_DEV, N // 2), jnp.int8),
            pltpu.VMEM((2, M // N_DEV, N // 2), jnp.int8),
            pltpu.VMEM((N_DEV, 128), jnp.float32),
            pltpu.VMEM((1, 128), jnp.float32),
            pltpu.SemaphoreType.DMA((2,)),
            pltpu.SemaphoreType.DMA((2,)),
            pltpu.SemaphoreType.DMA((2,)),
            pltpu.SemaphoreType.DMA((2,)),
            pltpu.SemaphoreType.DMA((N_DEV,)),
            pltpu.SemaphoreType.DMA((N_DEV,)),
            pltpu.SemaphoreType.DMA((4,)),
            pltpu.SemaphoreType.REGULAR,
            pltpu.SemaphoreType.REGULAR,
        ],
        compiler_params=pltpu.CompilerParams(
            collective_id=0,
            vmem_limit_bytes=64 * 1024 * 1024,
        ),
    )(x, w_mat)


# device time: 997848 ns/iter; 1.8702x vs baseline; 1.0747x over previous
import jax
import jax.numpy as jnp
from jax import lax
from jax.experimental import pallas as pl
from jax.experimental.pallas import tpu as pltpu

N_DEV = 8
NPASS = 2


def kernel(x, w_mat):
    M, Kp = x.shape
    _, N = w_mat.shape
    CH = M // N_DEV
    NH = N // NPASS
    HW = NH // 2
    QW = N // 4
    AGW = N // 2

    def body(x_ref, w_ref, out_ref,
             bcw, bccw, ccw_i8, cccw_i8, amax_rbuf, amax_sbuf,
             snd_cw, rcv_cw, snd_ccw, rcv_ccw,
             a_snd, a_rcv, dma_sems, cred_cw, cred_ccw):
        d = lax.axis_index("i")
        right = (d + 1) % N_DEV
        left = (d + N_DEV - 1) % N_DEV

        barrier = pltpu.get_barrier_semaphore()
        for nbr in (left, right):
            pl.semaphore_signal(barrier, 1, device_id=(nbr,))
        pl.semaphore_wait(barrier, 2)

        amax_rbuf[...] = jnp.zeros((N_DEV, 128), jnp.float32)

        def gemm_cols(cidx, c0, width):
            return jnp.dot(
                x_ref[pl.ds(cidx * CH, CH), :],
                w_ref[:, c0:c0 + width],
                preferred_element_type=jnp.float32,
            )

        def hop_start(buf_cw, buf_ccw, s_slot, r_slot, first):
            if not first:
                pl.semaphore_wait(cred_cw, 1)
                pl.semaphore_wait(cred_ccw, 1)
            r1 = pltpu.make_async_remote_copy(
                src_ref=buf_cw.at[s_slot], dst_ref=buf_cw.at[r_slot],
                send_sem=snd_cw.at[s_slot], recv_sem=rcv_cw.at[r_slot],
                device_id=(right,),
            )
            r2 = pltpu.make_async_remote_copy(
                src_ref=buf_ccw.at[s_slot], dst_ref=buf_ccw.at[r_slot],
                send_sem=snd_ccw.at[s_slot], recv_sem=rcv_ccw.at[r_slot],
                device_id=(left,),
            )
            r1.start()
            r2.start()
            return r1, r2

        def credits():
            pl.semaphore_signal(cred_cw, 1, device_id=(left,))
            pl.semaphore_signal(cred_ccw, 1, device_id=(right,))

        def out_dma(buf, slot, rows_at, c0, width, sem_i):
            return pltpu.make_async_copy(
                buf.at[slot],
                out_ref.at[pl.ds(rows_at * CH, CH), pl.ds(c0, width)],
                dma_sems.at[sem_i],
            )

        amax_local = jnp.float32(0.0)
        k = 0
        bcw[0] = gemm_cols((d + N_DEV - 1) % N_DEV, 0, HW)
        bccw[0] = gemm_cols((d + 1) % N_DEV, HW, HW)
        for p in range(NPASS):
            cw0 = p * NH
            ccw0 = p * NH + HW
            for h in range(N_DEV - 1):
                s_slot, r_slot = k % 2, (k + 1) % 2
                r1, r2 = hop_start(bcw, bccw, s_slot, r_slot,
                                   first=(k == 0))
                if h < N_DEV - 2:
                    c_cw = (d + 2 * N_DEV - h - 2) % N_DEV
                    c_ccw = (d + h + 2) % N_DEV
                else:
                    c_cw = c_ccw = d
                g1 = gemm_cols(c_cw, cw0, HW)
                g2 = gemm_cols(c_ccw, ccw0, HW)
                r1.wait()
                r2.wait()
                if h < N_DEV - 2:
                    bcw[r_slot] = bcw[r_slot] + g1
                    bccw[r_slot] = bccw[r_slot] + g2
                else:
                    fcw = jnp.maximum(bcw[r_slot] + g1, 0.0)
                    fccw = jnp.maximum(bccw[r_slot] + g2, 0.0)
                    bcw[r_slot] = fcw
                    bccw[r_slot] = fccw
                    amax_local = jnp.maximum(
                        amax_local,
                        jnp.maximum(jnp.max(fcw), jnp.max(fccw)))
                    cp1 = out_dma(bcw, r_slot, d, cw0, HW, 0)
                    cp2 = out_dma(bccw, r_slot, d, ccw0, HW, 1)
                    cp1.start()
                    cp2.start()
                    cp1.wait()
                    cp2.wait()
                    if p < NPASS - 1:
                        bcw[r_slot] = gemm_cols((d + N_DEV - 1) % N_DEV,
                                                (p + 1) * NH, HW)
                        bccw[r_slot] = gemm_cols((d + 1) % N_DEV,
                                                 (p + 1) * NH + HW, HW)
                credits()
                k += 1

        amax_sbuf[...] = jnp.full((1, 128), amax_local, jnp.float32)
        sends = []
        for off in range(1, N_DEV):
            tgt = (d + off) % N_DEV
            rd = pltpu.make_async_remote_copy(
                src_ref=amax_sbuf,
                dst_ref=amax_rbuf.at[pl.ds(d, 1)],
                send_sem=a_snd.at[off],
                recv_sem=a_rcv.at[d],
                device_id=(tgt,),
            )
            rd.start()
            sends.append(rd)
        for off in range(1, N_DEV):
            src = (d + off) % N_DEV
            rr = pltpu.make_async_remote_copy(
                src_ref=amax_sbuf,
                dst_ref=amax_rbuf.at[pl.ds(src, 1)],
                send_sem=a_snd.at[0],
                recv_sem=a_rcv.at[src],
                device_id=(left,),
            )
            rr.wait_recv()
        for rd in sends:
            rd.wait_send()
        amax_g = jnp.maximum(jnp.max(amax_rbuf[...]), amax_local)
        inv = 127.0 / amax_g
        scale = amax_g / 127.0

        quarters = ((bcw, 0), (bcw, 1), (bccw, 0), (bccw, 1))
        for qi, (buf, slot) in enumerate(quarters):
            cp = pltpu.make_async_copy(
                out_ref.at[pl.ds(d * CH, CH), pl.ds(qi * QW, QW)],
                buf.at[slot],
                dma_sems.at[qi],
            )
            cp.start()
            cp.wait()
            q = jnp.clip(jnp.round(buf[slot] * inv), -127.0, 127.0)
            cref, off = (ccw_i8, qi) if qi < 2 else (cccw_i8, qi - 2)
            cref[0, :, off * QW:(off + 1) * QW] = q.astype(jnp.int8)
        def dequant_store(slot, o_cw, o_ccw):
            dmas = []
            for qi, (buf, bslot) in enumerate(quarters):
                cref, off = (ccw_i8, qi) if qi < 2 else (cccw_i8, qi - 2)
                origin = o_cw if qi < 2 else o_ccw
                buf[bslot] = (
                    cref[slot, :, off * QW:(off + 1) * QW]
                    .astype(jnp.float32) * scale
                )
                cp = out_dma(buf, bslot, origin, qi * QW, QW, qi)
                cp.start()
                dmas.append(cp)
            for cp in dmas:
                cp.wait()

        for a in range(N_DEV - 1):
            s_slot, r_slot = a % 2, (a + 1) % 2
            r1, r2 = hop_start(ccw_i8, cccw_i8, s_slot, r_slot,
                               first=False)
            dequant_store(s_slot,
                          (d + 2 * N_DEV - a) % N_DEV,
                          (d + a) % N_DEV)
            r1.wait()
            r2.wait()
            credits()
        dequant_store((N_DEV - 1) % 2,
                      (d + N_DEV + 1) % N_DEV,
                      (d + N_DEV - 1) % N_DEV)
        pl.semaphore_wait(cred_cw, 1)
        pl.semaphore_wait(cred_ccw, 1)

    return pl.pallas_call(
        body,
        out_shape=jax.ShapeDtypeStruct((M, N), jnp.float32),
        in_specs=[
            pl.BlockSpec(memory_space=pltpu.VMEM),
            pl.BlockSpec(memory_space=pltpu.VMEM),
        ],
        out_specs=pl.BlockSpec(memory_space=pl.ANY),
        scratch_shapes=[
            pltpu.VMEM((2, M // N_DEV, NH // 2), jnp.float32),
            pltpu.VMEM((2, M // N_DEV, NH // 2), jnp.float32),
            pltpu.VMEM((2, M // N_DEV, N // 2), jnp.int8),
            pltpu.VMEM((2, M // N_DEV, N // 2), jnp.int8),
            pltpu.VMEM((N_DEV, 128), jnp.float32),
            pltpu.VMEM((1, 128), jnp.float32),
            pltpu.SemaphoreType.DMA((2,)),
            pltpu.SemaphoreType.DMA((2,)),
            pltpu.SemaphoreType.DMA((2,)),
            pltpu.SemaphoreType.DMA((2,)),
            pltpu.SemaphoreType.DMA((N_DEV,)),
            pltpu.SemaphoreType.DMA((N_DEV,)),
            pltpu.SemaphoreType.DMA((4,)),
            pltpu.SemaphoreType.REGULAR,
            pltpu.SemaphoreType.REGULAR,
        ],
        compiler_params=pltpu.CompilerParams(
            collective_id=0,
            vmem_limit_bytes=64 * 1024 * 1024,
        ),
    )(x, w_mat)


# device time: 991645 ns/iter; 1.8819x vs baseline; 1.0063x over previous
import jax
import jax.numpy as jnp
from jax import lax
from jax.experimental import pallas as pl
from jax.experimental.pallas import tpu as pltpu

N_DEV = 8
NPASS = 2


def kernel(x, w_mat):
    M, Kp = x.shape
    _, N = w_mat.shape
    CH = M // N_DEV
    NH = N // NPASS
    HW = NH // 2
    QW = N // 4
    AGW = N // 2

    def body(x_ref, w_ref, out_ref,
             bcw, bccw, ccw_i8, cccw_i8, amax_rbuf, amax_sbuf,
             snd_cw, rcv_cw, snd_ccw, rcv_ccw,
             a_snd, a_rcv, dma_sems, cred_cw, cred_ccw):
        d = lax.axis_index("i")
        right = (d + 1) % N_DEV
        left = (d + N_DEV - 1) % N_DEV

        barrier = pltpu.get_barrier_semaphore()
        for nbr in (left, right):
            pl.semaphore_signal(barrier, 1, device_id=(nbr,))
        pl.semaphore_wait(barrier, 2)

        amax_rbuf[...] = jnp.zeros((N_DEV, 128), jnp.float32)

        def gemm_cols(cidx, c0, width):
            return jnp.dot(
                x_ref[pl.ds(cidx * CH, CH), :],
                w_ref[:, c0:c0 + width],
                preferred_element_type=jnp.float32,
            )

        def hop_start(buf_cw, buf_ccw, s_slot, r_slot, first):
            if not first:
                pl.semaphore_wait(cred_cw, 1)
                pl.semaphore_wait(cred_ccw, 1)
            r1 = pltpu.make_async_remote_copy(
                src_ref=buf_cw.at[s_slot], dst_ref=buf_cw.at[r_slot],
                send_sem=snd_cw.at[s_slot], recv_sem=rcv_cw.at[r_slot],
                device_id=(right,),
            )
            r2 = pltpu.make_async_remote_copy(
                src_ref=buf_ccw.at[s_slot], dst_ref=buf_ccw.at[r_slot],
                send_sem=snd_ccw.at[s_slot], recv_sem=rcv_ccw.at[r_slot],
                device_id=(left,),
            )
            r1.start()
            r2.start()
            return r1, r2

        def credits():
            pl.semaphore_signal(cred_cw, 1, device_id=(left,))
            pl.semaphore_signal(cred_ccw, 1, device_id=(right,))

        def out_dma(buf, slot, rows_at, c0, width, sem_i):
            return pltpu.make_async_copy(
                buf.at[slot],
                out_ref.at[pl.ds(rows_at * CH, CH), pl.ds(c0, width)],
                dma_sems.at[sem_i],
            )

        amax_local = jnp.float32(0.0)
        k = 0
        bcw[0] = gemm_cols((d + N_DEV - 1) % N_DEV, 0, HW)
        bccw[0] = gemm_cols((d + 1) % N_DEV, HW, HW)
        for p in range(NPASS):
            cw0 = p * NH
            ccw0 = p * NH + HW
            for h in range(N_DEV - 1):
                s_slot, r_slot = k % 2, (k + 1) % 2
                r1, r2 = hop_start(bcw, bccw, s_slot, r_slot,
                                   first=(k == 0))
                if h < N_DEV - 2:
                    c_cw = (d + 2 * N_DEV - h - 2) % N_DEV
                    c_ccw = (d + h + 2) % N_DEV
                else:
                    c_cw = c_ccw = d
                g1 = gemm_cols(c_cw, cw0, HW)
                g2 = gemm_cols(c_ccw, ccw0, HW)
                r1.wait()
                r2.wait()
                if h < N_DEV - 2:
                    bcw[r_slot] = bcw[r_slot] + g1
                    bccw[r_slot] = bccw[r_slot] + g2
                else:
                    fcw = jnp.maximum(bcw[r_slot] + g1, 0.0)
                    fccw = jnp.maximum(bccw[r_slot] + g2, 0.0)
                    bcw[r_slot] = fcw
                    bccw[r_slot] = fccw
                    amax_local = jnp.maximum(
                        amax_local,
                        jnp.maximum(jnp.max(fcw), jnp.max(fccw)))
                    cp1 = out_dma(bcw, r_slot, d, cw0, HW, 0)
                    cp2 = out_dma(bccw, r_slot, d, ccw0, HW, 1)
                    cp1.start()
                    cp2.start()
                    cp1.wait()
                    cp2.wait()
                    if p < NPASS - 1:
                        bcw[r_slot] = gemm_cols((d + N_DEV - 1) % N_DEV,
                                                (p + 1) * NH, HW)
                        bccw[r_slot] = gemm_cols((d + 1) % N_DEV,
                                                 (p + 1) * NH + HW, HW)
                credits()
                k += 1

        amax_sbuf[...] = jnp.full((1, 128), amax_local, jnp.float32)
        sends = []
        for off in range(1, N_DEV):
            tgt = (d + off) % N_DEV
            rd = pltpu.make_async_remote_copy(
                src_ref=amax_sbuf,
                dst_ref=amax_rbuf.at[pl.ds(d, 1)],
                send_sem=a_snd.at[off],
                recv_sem=a_rcv.at[d],
                device_id=(tgt,),
            )
            rd.start()
            sends.append(rd)
        quarters = ((bcw, 0), (bcw, 1), (bccw, 0), (bccw, 1))
        rb = []
        for qi, (buf, slot) in enumerate(quarters):
            cp = pltpu.make_async_copy(
                out_ref.at[pl.ds(d * CH, CH), pl.ds(qi * QW, QW)],
                buf.at[slot],
                dma_sems.at[qi],
            )
            cp.start()
            rb.append(cp)
        for off in range(1, N_DEV):
            src = (d + off) % N_DEV
            rr = pltpu.make_async_remote_copy(
                src_ref=amax_sbuf,
                dst_ref=amax_rbuf.at[pl.ds(src, 1)],
                send_sem=a_snd.at[0],
                recv_sem=a_rcv.at[src],
                device_id=(left,),
            )
            rr.wait_recv()
        for rd in sends:
            rd.wait_send()
        amax_g = jnp.maximum(jnp.max(amax_rbuf[...]), amax_local)
        inv = 127.0 / amax_g
        scale = amax_g / 127.0

        for qi, (buf, slot) in enumerate(quarters):
            rb[qi].wait()
            q = jnp.clip(jnp.round(buf[slot] * inv), -127.0, 127.0)
            cref, off = (ccw_i8, qi) if qi < 2 else (cccw_i8, qi - 2)
            cref[0, :, off * QW:(off + 1) * QW] = q.astype(jnp.int8)
        def dequant_store(slot, o_cw, o_ccw):
            dmas = []
            for qi, (buf, bslot) in enumerate(quarters):
                cref, off = (ccw_i8, qi) if qi < 2 else (cccw_i8, qi - 2)
                origin = o_cw if qi < 2 else o_ccw
                buf[bslot] = (
                    cref[slot, :, off * QW:(off + 1) * QW]
                    .astype(jnp.float32) * scale
                )
                cp = out_dma(buf, bslot, origin, qi * QW, QW, qi)
                cp.start()
                dmas.append(cp)
            for cp in dmas:
                cp.wait()

        for a in range(N_DEV - 1):
            s_slot, r_slot = a % 2, (a + 1) % 2
            r1, r2 = hop_start(ccw_i8, cccw_i8, s_slot, r_slot,
                               first=False)
            dequant_store(s_slot,
                          (d + 2 * N_DEV - a) % N_DEV,
                          (d + a) % N_DEV)
            r1.wait()
            r2.wait()
            credits()
        dequant_store((N_DEV - 1) % 2,
                      (d + N_DEV + 1) % N_DEV,
                      (d + N_DEV - 1) % N_DEV)
        pl.semaphore_wait(cred_cw, 1)
        pl.semaphore_wait(cred_ccw, 1)

    return pl.pallas_call(
        body,
        out_shape=jax.ShapeDtypeStruct((M, N), jnp.float32),
        in_specs=[
            pl.BlockSpec(memory_space=pltpu.VMEM),
            pl.BlockSpec(memory_space=pltpu.VMEM),
        ],
        out_specs=pl.BlockSpec(memory_space=pl.ANY),
        scratch_shapes=[
            pltpu.VMEM((2, M // N_DEV, NH // 2), jnp.float32),
            pltpu.VMEM((2, M // N_DEV, NH // 2), jnp.float32),
            pltpu.VMEM((2, M // N_DEV, N // 2), jnp.int8),
            pltpu.VMEM((2, M // N_DEV, N // 2), jnp.int8),
            pltpu.VMEM((N_DEV, 128), jnp.float32),
            pltpu.VMEM((1, 128), jnp.float32),
            pltpu.SemaphoreType.DMA((2,)),
            pltpu.SemaphoreType.DMA((2,)),
            pltpu.SemaphoreType.DMA((2,)),
            pltpu.SemaphoreType.DMA((2,)),
            pltpu.SemaphoreType.DMA((N_DEV,)),
            pltpu.SemaphoreType.DMA((N_DEV,)),
            pltpu.SemaphoreType.DMA((4,)),
            pltpu.SemaphoreType.REGULAR,
            pltpu.SemaphoreType.REGULAR,
        ],
        compiler_params=pltpu.CompilerParams(
            collective_id=0,
            vmem_limit_bytes=64 * 1024 * 1024,
        ),
    )(x, w_mat)


# device time: 980452 ns/iter; 1.9033x vs baseline; 1.0114x over previous
import jax
import jax.numpy as jnp
from jax import lax
from jax.experimental import pallas as pl
from jax.experimental.pallas import tpu as pltpu

N_DEV = 8
NPASS = 2


def kernel(x, w_mat):
    M, Kp = x.shape
    _, N = w_mat.shape
    CH = M // N_DEV
    NH = N // NPASS
    HW = NH // 2
    QW = N // 4
    AGW = N // 2

    def body(x_ref, w_ref, out_ref,
             bcw, bccw, ccw_i8, cccw_i8, amax_rbuf, amax_sbuf,
             snd_cw, rcv_cw, snd_ccw, rcv_ccw,
             a_snd, a_rcv, dma_sems, cred_cw, cred_ccw):
        d = lax.axis_index("i")
        right = (d + 1) % N_DEV
        left = (d + N_DEV - 1) % N_DEV

        barrier = pltpu.get_barrier_semaphore()
        for nbr in (left, right):
            pl.semaphore_signal(barrier, 1, device_id=(nbr,))
        pl.semaphore_wait(barrier, 2)

        amax_rbuf[...] = jnp.zeros((N_DEV, 128), jnp.float32)

        def gemm_cols(cidx, c0, width):
            return jnp.dot(
                x_ref[pl.ds(cidx * CH, CH), :],
                w_ref[:, c0:c0 + width],
                preferred_element_type=jnp.float32,
            )

        def hop_start(buf_cw, buf_ccw, s_slot, r_slot, first):
            if not first:
                pl.semaphore_wait(cred_cw, 1)
                pl.semaphore_wait(cred_ccw, 1)
            r1 = pltpu.make_async_remote_copy(
                src_ref=buf_cw.at[s_slot], dst_ref=buf_cw.at[r_slot],
                send_sem=snd_cw.at[s_slot], recv_sem=rcv_cw.at[r_slot],
                device_id=(right,),
            )
            r2 = pltpu.make_async_remote_copy(
                src_ref=buf_ccw.at[s_slot], dst_ref=buf_ccw.at[r_slot],
                send_sem=snd_ccw.at[s_slot], recv_sem=rcv_ccw.at[r_slot],
                device_id=(left,),
            )
            r1.start()
            r2.start()
            return r1, r2

        def credits():
            pl.semaphore_signal(cred_cw, 1, device_id=(left,))
            pl.semaphore_signal(cred_ccw, 1, device_id=(right,))

        def out_dma(buf, slot, rows_at, c0, width, sem_i):
            return pltpu.make_async_copy(
                buf.at[slot],
                out_ref.at[pl.ds(rows_at * CH, CH), pl.ds(c0, width)],
                dma_sems.at[sem_i],
            )

        amax_local = jnp.float32(0.0)
        k = 0
        bcw[0] = gemm_cols((d + N_DEV - 1) % N_DEV, 0, HW)
        bccw[0] = gemm_cols((d + 1) % N_DEV, HW, HW)
        for p in range(NPASS):
            cw0 = p * NH
            ccw0 = p * NH + HW
            for h in range(N_DEV - 1):
                s_slot, r_slot = k % 2, (k + 1) % 2
                r1, r2 = hop_start(bcw, bccw, s_slot, r_slot,
                                   first=(k == 0))
                if h < N_DEV - 2:
                    c_cw = (d + 2 * N_DEV - h - 2) % N_DEV
                    c_ccw = (d + h + 2) % N_DEV
                else:
                    c_cw = c_ccw = d
                g1 = gemm_cols(c_cw, cw0, HW)
                g2 = gemm_cols(c_ccw, ccw0, HW)
                r1.wait()
                r2.wait()
                credits()
                if h < N_DEV - 2:
                    bcw[r_slot] = bcw[r_slot] + g1
                    bccw[r_slot] = bccw[r_slot] + g2
                else:
                    fcw = jnp.maximum(bcw[r_slot] + g1, 0.0)
                    fccw = jnp.maximum(bccw[r_slot] + g2, 0.0)
                    bcw[r_slot] = fcw
                    bccw[r_slot] = fccw
                    amax_local = jnp.maximum(
                        amax_local,
                        jnp.maximum(jnp.max(fcw), jnp.max(fccw)))
                    cp1 = out_dma(bcw, r_slot, d, cw0, HW, 0)
                    cp2 = out_dma(bccw, r_slot, d, ccw0, HW, 1)
                    cp1.start()
                    cp2.start()
                    cp1.wait()
                    cp2.wait()
                    if p < NPASS - 1:
                        bcw[r_slot] = gemm_cols((d + N_DEV - 1) % N_DEV,
                                                (p + 1) * NH, HW)
                        bccw[r_slot] = gemm_cols((d + 1) % N_DEV,
                                                 (p + 1) * NH + HW, HW)
                k += 1

        amax_sbuf[...] = jnp.full((1, 128), amax_local, jnp.float32)
        sends = []
        for off in range(1, N_DEV):
            tgt = (d + off) % N_DEV
            rd = pltpu.make_async_remote_copy(
                src_ref=amax_sbuf,
                dst_ref=amax_rbuf.at[pl.ds(d, 1)],
                send_sem=a_snd.at[off],
                recv_sem=a_rcv.at[d],
                device_id=(tgt,),
            )
            rd.start()
            sends.append(rd)
        quarters = ((bcw, 0), (bcw, 1), (bccw, 0), (bccw, 1))
        rb = []
        for qi, (buf, slot) in enumerate(quarters):
            cp = pltpu.make_async_copy(
                out_ref.at[pl.ds(d * CH, CH), pl.ds(qi * QW, QW)],
                buf.at[slot],
                dma_sems.at[qi],
            )
            cp.start()
            rb.append(cp)
        for off in range(1, N_DEV):
            src = (d + off) % N_DEV
            rr = pltpu.make_async_remote_copy(
                src_ref=amax_sbuf,
                dst_ref=amax_rbuf.at[pl.ds(src, 1)],
                send_sem=a_snd.at[0],
                recv_sem=a_rcv.at[src],
                device_id=(left,),
            )
            rr.wait_recv()
        for rd in sends:
            rd.wait_send()
        amax_g = jnp.maximum(jnp.max(amax_rbuf[...]), amax_local)
        inv = 127.0 / amax_g
        scale = amax_g / 127.0

        for qi, (buf, slot) in enumerate(quarters):
            rb[qi].wait()
            q = jnp.clip(jnp.round(buf[slot] * inv), -127.0, 127.0)
            cref, off = (ccw_i8, qi) if qi < 2 else (cccw_i8, qi - 2)
            cref[0, :, off * QW:(off + 1) * QW] = q.astype(jnp.int8)
        def dequant_store(slot, o_cw, o_ccw):
            dmas = []
            for qi, (buf, bslot) in enumerate(quarters):
                cref, off = (ccw_i8, qi) if qi < 2 else (cccw_i8, qi - 2)
                origin = o_cw if qi < 2 else o_ccw
                buf[bslot] = (
                    cref[slot, :, off * QW:(off + 1) * QW]
                    .astype(jnp.float32) * scale
                )
                cp = out_dma(buf, bslot, origin, qi * QW, QW, qi)
                cp.start()
                dmas.append(cp)
            for cp in dmas:
                cp.wait()

        for a in range(N_DEV - 1):
            s_slot, r_slot = a % 2, (a + 1) % 2
            r1, r2 = hop_start(ccw_i8, cccw_i8, s_slot, r_slot,
                               first=False)
            dequant_store(s_slot,
                          (d + 2 * N_DEV - a) % N_DEV,
                          (d + a) % N_DEV)
            r1.wait()
            r2.wait()
            credits()
        dequant_store((N_DEV - 1) % 2,
                      (d + N_DEV + 1) % N_DEV,
                      (d + N_DEV - 1) % N_DEV)
        pl.semaphore_wait(cred_cw, 1)
        pl.semaphore_wait(cred_ccw, 1)

    return pl.pallas_call(
        body,
        out_shape=jax.ShapeDtypeStruct((M, N), jnp.float32),
        in_specs=[
            pl.BlockSpec(memory_space=pltpu.VMEM),
            pl.BlockSpec(memory_space=pltpu.VMEM),
        ],
        out_specs=pl.BlockSpec(memory_space=pl.ANY),
        scratch_shapes=[
            pltpu.VMEM((2, M // N_DEV, NH // 2), jnp.float32),
            pltpu.VMEM((2, M // N_DEV, NH // 2), jnp.float32),
            pltpu.VMEM((2, M // N_DEV, N // 2), jnp.int8),
            pltpu.VMEM((2, M // N_DEV, N // 2), jnp.int8),
            pltpu.VMEM((N_DEV, 128), jnp.float32),
            pltpu.VMEM((1, 128), jnp.float32),
            pltpu.SemaphoreType.DMA((2,)),
            pltpu.SemaphoreType.DMA((2,)),
            pltpu.SemaphoreType.DMA((2,)),
            pltpu.SemaphoreType.DMA((2,)),
            pltpu.SemaphoreType.DMA((N_DEV,)),
            pltpu.SemaphoreType.DMA((N_DEV,)),
            pltpu.SemaphoreType.DMA((4,)),
            pltpu.SemaphoreType.REGULAR,
            pltpu.SemaphoreType.REGULAR,
        ],
        compiler_params=pltpu.CompilerParams(
            collective_id=0,
            vmem_limit_bytes=64 * 1024 * 1024,
        ),
    )(x, w_mat)
